# baseline (speedup 1.0000x reference)
import jax
import jax.numpy as jnp
from jax.experimental import pallas as pl
from jax.experimental.pallas import tpu as pltpu

BC = 8


def _softmax(logits):
    m = jnp.max(logits, axis=-1, keepdims=True)
    e = jnp.exp(logits - m)
    return e / jnp.sum(e, axis=-1, keepdims=True)


def _half(xg, w0_ref, be0_ref, br0_ref,
          w1_ref, be1_ref, br1_ref,
          f0w_ref, f0be_ref, f0wr_ref, f0br_ref,
          f1w_ref, f1be_ref, f1wr_ref, f1br_ref):
    bc = xg.shape[0]
    E = 4
    M = bc * 256
    pieces = [xg[:, di:di + 32, :, :].reshape(M, 18) for di in range(3)]
    pm = jnp.concatenate(pieces, axis=-1)
    y = jnp.dot(pm, w0_ref[...], preferred_element_type=jnp.float32)
    y4 = y.reshape(bc, 256, 2048)
    rs = (y4[:, :, 482:486] + y4[:, :, 994:998]
          + y4[:, :, 1506:1510] + y4[:, :, 2018:2022])
    logits = jnp.mean(rs, axis=1) * 0.25 + br0_ref[...]
    g = _softmax(logits)
    be0 = be0_ref[...]
    mixes = []
    for p in range(4):
        mp = jnp.zeros((bc, 256, 128), jnp.float32)
        for e in range(E):
            sl = slice(512 * p + e * 128, 512 * p + (e + 1) * 128)
            bias = be0[:, e * 128:(e + 1) * 128][None]
            mp += g[:, e:e + 1, None] * jax.nn.relu(y4[:, :, sl] + bias)
        mixes.append(mp)
    ev = jnp.maximum(mixes[0], mixes[1])
    od = jnp.maximum(mixes[2], mixes[3])
    ev = jnp.max(ev.reshape(bc, 16, 2, 8, 128), axis=2)
    od = jnp.max(od.reshape(bc, 16, 2, 8, 128), axis=2)
    ev = ev.astype(jnp.bfloat16)
    od = od.astype(jnp.bfloat16)

    zjg = jnp.zeros((bc, 16, 1, 128), jnp.bfloat16)
    odm = jnp.concatenate([zjg, od[:, :, :7, :]], axis=2)
    evp = jnp.concatenate([ev[:, :, 1:, :], zjg], axis=2)
    bases = [jnp.stack(pair, axis=2)
             for pair in ((odm, ev), (ev, od), (od, evp))]
    zrow = jnp.zeros((bc, 1, 2, 8, 128), jnp.bfloat16)
    CO = 192
    pieces1 = []
    for di in range(3):
        for dj in range(3):
            b = bases[dj]
            if di == 0:
                piece = jnp.concatenate([zrow, b[:, 0:15]], axis=1)
            elif di == 1:
                piece = b
            else:
                piece = jnp.concatenate([b[:, 1:16], zrow], axis=1)
            pieces1.append(piece.reshape(M, 128))
    pm1 = jnp.concatenate(pieces1, axis=-1)
    y1 = jnp.dot(pm1, w1_ref[...], preferred_element_type=jnp.float32)
    y3 = y1.reshape(bc, 256, E * CO + E)
    logits1 = jnp.mean(y3[:, :, E * CO:], axis=1) + br1_ref[...]
    g1 = _softmax(logits1)
    be1 = be1_ref[...]
    mixed = jnp.zeros((bc, 256, CO), jnp.float32)
    for e in range(E):
        ye = y3[:, :, e * CO:(e + 1) * CO] + be1[:, e * CO:(e + 1) * CO][None]
        mixed += g1[:, e:e + 1, None] * jax.nn.relu(ye)
    hm = jnp.max(mixed.reshape(bc, 8, 2, 2, 8, CO), axis=(2, 3))
    h = jnp.mean(hm.reshape(bc, 64, CO), axis=1)

    hb = h.astype(jnp.bfloat16)
    g0 = _softmax(jnp.dot(hb, f0wr_ref[...],
                          preferred_element_type=jnp.float32) + f0br_ref[...])
    o0 = jax.nn.relu(jnp.dot(hb, f0w_ref[...],
                             preferred_element_type=jnp.float32) + f0be_ref[...])
    h1 = jnp.zeros((bc, 146), jnp.float32)
    for e in range(E):
        h1 += g0[:, e:e + 1] * o0[:, e * 146:(e + 1) * 146]

    h1b = h1.astype(jnp.bfloat16)
    g1f = _softmax(jnp.dot(h1b, f1wr_ref[...],
                           preferred_element_type=jnp.float32) + f1br_ref[...])
    o1 = jax.nn.relu(jnp.dot(h1b, f1w_ref[...],
                             preferred_element_type=jnp.float32) + f1be_ref[...])
    h2 = jnp.zeros((bc, 100), jnp.float32)
    for e in range(E):
        h2 += g1f[:, e:e + 1] * o1[:, e * 100:(e + 1) * 100]
    return h2


def _net_kernel(xg_ref, w0_ref, be0_ref, br0_ref,
                w1_ref, be1_ref, br1_ref,
                f0w_ref, f0be_ref, f0wr_ref, f0br_ref,
                f1w_ref, f1be_ref, f1wr_ref, f1br_ref,
                out_ref):
    wargs = (w0_ref, be0_ref, br0_ref, w1_ref, be1_ref, br1_ref,
             f0w_ref, f0be_ref, f0wr_ref, f0br_ref,
             f1w_ref, f1be_ref, f1wr_ref, f1br_ref)
    half = BC // 2
    out_ref[0:half] = _half(xg_ref[0:half], *wargs)
    out_ref[half:BC] = _half(xg_ref[half:BC], *wargs)


def kernel(x, cnn0_We, cnn0_be, cnn0_Wr, cnn0_br,
           cnn1_We, cnn1_be, cnn1_Wr, cnn1_br,
           fc0_We, fc0_be, fc0_Wr, fc0_br,
           fc1_We, fc1_be, fc1_Wr, fc1_br):
    B = x.shape[0]
    xh = jnp.transpose(x, (0, 2, 3, 1))
    xp = jnp.pad(xh, ((0, 0), (1, 1), (1, 1), (0, 0))).astype(jnp.bfloat16)
    xg = jnp.stack([xp[:, :, 4 * j:4 * j + 6, :] for j in range(8)], axis=2)
    xg = xg.reshape(B, 34, 8, 18)
    w0e = jnp.transpose(cnn0_We, (3, 4, 2, 0, 1))
    blk = jnp.pad(w0e, ((0, 0),) * 4 + ((0, 30),)).reshape(3, 3, 3, 512)
    w0r = jnp.transpose(cnn0_Wr, (2, 3, 1, 0))
    blk = blk.at[:, :, :, 482:486].set(w0r)
    w0 = jnp.zeros((3, 6, 3, 2048), jnp.float32)
    for p in range(4):
        w0 = w0.at[:, p:p + 3, :, 512 * p:512 * (p + 1)].set(blk)
    w0 = w0.reshape(54, 2048).astype(jnp.bfloat16)
    be0 = jnp.pad(cnn0_be, ((0, 0), (0, 30))).reshape(1, 512)
    br0 = cnn0_br.reshape(1, 4)
    w1e = jnp.transpose(cnn1_We, (3, 4, 2, 0, 1)).reshape(9, 98, 768)
    w1r = jnp.transpose(cnn1_Wr, (2, 3, 1, 0)).reshape(9, 98, 4)
    w1 = jnp.concatenate([w1e, w1r], axis=-1)
    w1 = jnp.pad(w1, ((0, 0), (0, 30), (0, 0)))
    w1 = w1.reshape(9 * 128, 772).astype(jnp.bfloat16)
    be1 = cnn1_be.reshape(1, 768)
    br1 = cnn1_br.reshape(1, 4)
    f0w = jnp.transpose(fc0_We, (1, 0, 2)).reshape(192, 584).astype(jnp.bfloat16)
    f0be = fc0_be.reshape(1, 584)
    f0wr = fc0_Wr.astype(jnp.bfloat16)
    f0br = fc0_br.reshape(1, 4)
    f1w = jnp.transpose(fc1_We, (1, 0, 2)).reshape(146, 400).astype(jnp.bfloat16)
    f1be = fc1_be.reshape(1, 400)
    f1wr = fc1_Wr.astype(jnp.bfloat16)
    f1br = fc1_br.reshape(1, 4)

    full = lambda i: (0, 0)
    out = pl.pallas_call(
        _net_kernel,
        grid=(B // BC,),
        in_specs=[
            pl.BlockSpec((BC, 34, 8, 18), lambda i: (i, 0, 0, 0)),
            pl.BlockSpec((54, 2048), full),
            pl.BlockSpec((1, 512), full),
            pl.BlockSpec((1, 4), full),
            pl.BlockSpec((1152, 772), full),
            pl.BlockSpec((1, 768), full),
            pl.BlockSpec((1, 4), full),
            pl.BlockSpec((192, 584), full),
            pl.BlockSpec((1, 584), full),
            pl.BlockSpec((192, 4), full),
            pl.BlockSpec((1, 4), full),
            pl.BlockSpec((146, 400), full),
            pl.BlockSpec((1, 400), full),
            pl.BlockSpec((146, 4), full),
            pl.BlockSpec((1, 4), full),
        ],
        out_specs=pl.BlockSpec((BC, 100), lambda i: (i, 0)),
        out_shape=jax.ShapeDtypeStruct((B, 100), jnp.float32),
        compiler_params=pltpu.CompilerParams(
            dimension_semantics=("arbitrary",)),
    )(xg, w0, be0, br0, w1, be1, br1,
      f0w, f0be, f0wr, f0br, f1w, f1be, f1wr, f1br)
    return out


# speedup vs baseline: 1.0819x; 1.0819x over previous
import jax
import jax.numpy as jnp
from jax.experimental import pallas as pl
from jax.experimental.pallas import tpu as pltpu

BC = 16


def _softmax(logits):
    m = jnp.max(logits, axis=-1, keepdims=True)
    e = jnp.exp(logits - m)
    return e / jnp.sum(e, axis=-1, keepdims=True)


def _net_kernel(xg_ref, w0_ref, be0_ref, br0_ref,
                w1_ref, be1_ref, br1_ref,
                f0w_ref, f0be_ref, f0wr_ref, f0br_ref,
                f1w_ref, f1be_ref, f1wr_ref, f1br_ref,
                out_ref):
    E = 4
    pieces = [xg_ref[:, di:di + 32, :, :].reshape(BC * 32 * 8, 18)
              for di in range(3)]
    pm = jnp.concatenate(pieces, axis=-1)
    y = jnp.dot(pm, w0_ref[...], preferred_element_type=jnp.float32)
    y4 = y.reshape(BC, 256, 2048)
    rs = (y4[:, :, 482:486] + y4[:, :, 994:998]
          + y4[:, :, 1506:1510] + y4[:, :, 2018:2022])
    logits = jnp.mean(rs, axis=1) * 0.25 + br0_ref[...]
    g = _softmax(logits)
    be0 = be0_ref[...]
    mixes = []
    for p in range(4):
        mp = jnp.zeros((BC, 256, 128), jnp.float32)
        for e in range(E):
            sl = slice(512 * p + e * 128, 512 * p + (e + 1) * 128)
            bias = be0[:, e * 128:(e + 1) * 128][None]
            mp += g[:, e:e + 1, None] * jax.nn.relu(y4[:, :, sl] + bias)
        mixes.append(mp)
    ev = jnp.maximum(mixes[0], mixes[1])
    od = jnp.maximum(mixes[2], mixes[3])
    ev = jnp.max(ev.reshape(BC, 16, 2, 8, 128), axis=2)
    od = jnp.max(od.reshape(BC, 16, 2, 8, 128), axis=2)
    ev = ev.astype(jnp.bfloat16)
    od = od.astype(jnp.bfloat16)

    zjg = jnp.zeros((BC, 16, 1, 128), jnp.bfloat16)
    odm = jnp.concatenate([zjg, od[:, :, :7, :]], axis=2)
    evp = jnp.concatenate([ev[:, :, 1:, :], zjg], axis=2)
    bases = [jnp.stack(pair, axis=2)
             for pair in ((odm, ev), (ev, od), (od, evp))]
    zrow = jnp.zeros((BC, 1, 2, 8, 128), jnp.bfloat16)
    CO = 192
    pieces1 = []
    for di in range(3):
        for dj in range(3):
            b = bases[dj]
            if di == 0:
                piece = jnp.concatenate([zrow, b[:, 0:15]], axis=1)
            elif di == 1:
                piece = b
            else:
                piece = jnp.concatenate([b[:, 1:16], zrow], axis=1)
            pieces1.append(piece.reshape(BC * 256, 128))
    pm1 = jnp.concatenate(pieces1, axis=-1)
    y1 = jnp.dot(pm1, w1_ref[...], preferred_element_type=jnp.float32)
    y3 = y1.reshape(BC, 256, E * CO + E)
    logits1 = jnp.mean(y3[:, :, E * CO:], axis=1) + br1_ref[...]
    g1 = _softmax(logits1)
    be1 = be1_ref[...]
    mixed = jnp.zeros((BC, 256, CO), jnp.float32)
    for e in range(E):
        ye = y3[:, :, e * CO:(e + 1) * CO] + be1[:, e * CO:(e + 1) * CO][None]
        mixed += g1[:, e:e + 1, None] * jax.nn.relu(ye)
    hm = jnp.max(mixed.reshape(BC, 8, 2, 2, 8, CO), axis=(2, 3))
    h = jnp.mean(hm.reshape(BC, 64, CO), axis=1)

    hb = h.astype(jnp.bfloat16)
    g0 = _softmax(jnp.dot(hb, f0wr_ref[...],
                          preferred_element_type=jnp.float32) + f0br_ref[...])
    o0 = jax.nn.relu(jnp.dot(hb, f0w_ref[...],
                             preferred_element_type=jnp.float32) + f0be_ref[...])
    h1 = jnp.zeros((BC, 146), jnp.float32)
    for e in range(E):
        h1 += g0[:, e:e + 1] * o0[:, e * 146:(e + 1) * 146]

    h1b = h1.astype(jnp.bfloat16)
    g1f = _softmax(jnp.dot(h1b, f1wr_ref[...],
                           preferred_element_type=jnp.float32) + f1br_ref[...])
    o1 = jax.nn.relu(jnp.dot(h1b, f1w_ref[...],
                             preferred_element_type=jnp.float32) + f1be_ref[...])
    h2 = jnp.zeros((BC, 100), jnp.float32)
    for e in range(E):
        h2 += g1f[:, e:e + 1] * o1[:, e * 100:(e + 1) * 100]
    out_ref[...] = h2


def kernel(x, cnn0_We, cnn0_be, cnn0_Wr, cnn0_br,
           cnn1_We, cnn1_be, cnn1_Wr, cnn1_br,
           fc0_We, fc0_be, fc0_Wr, fc0_br,
           fc1_We, fc1_be, fc1_Wr, fc1_br):
    B = x.shape[0]
    xh = jnp.transpose(x, (0, 2, 3, 1))
    xp = jnp.pad(xh, ((0, 0), (1, 1), (1, 1), (0, 0))).astype(jnp.bfloat16)
    xg = jnp.stack([xp[:, :, 4 * j:4 * j + 6, :] for j in range(8)], axis=2)
    xg = xg.reshape(B, 34, 8, 18)
    w0e = jnp.transpose(cnn0_We, (3, 4, 2, 0, 1))
    blk = jnp.pad(w0e, ((0, 0),) * 4 + ((0, 30),)).reshape(3, 3, 3, 512)
    w0r = jnp.transpose(cnn0_Wr, (2, 3, 1, 0))
    blk = blk.at[:, :, :, 482:486].set(w0r)
    w0 = jnp.zeros((3, 6, 3, 2048), jnp.float32)
    for p in range(4):
        w0 = w0.at[:, p:p + 3, :, 512 * p:512 * (p + 1)].set(blk)
    w0 = w0.reshape(54, 2048).astype(jnp.bfloat16)
    be0 = jnp.pad(cnn0_be, ((0, 0), (0, 30))).reshape(1, 512)
    br0 = cnn0_br.reshape(1, 4)
    w1e = jnp.transpose(cnn1_We, (3, 4, 2, 0, 1)).reshape(9, 98, 768)
    w1r = jnp.transpose(cnn1_Wr, (2, 3, 1, 0)).reshape(9, 98, 4)
    w1 = jnp.concatenate([w1e, w1r], axis=-1)
    w1 = jnp.pad(w1, ((0, 0), (0, 30), (0, 0)))
    w1 = w1.reshape(9 * 128, 772).astype(jnp.bfloat16)
    be1 = cnn1_be.reshape(1, 768)
    br1 = cnn1_br.reshape(1, 4)
    f0w = jnp.transpose(fc0_We, (1, 0, 2)).reshape(192, 584).astype(jnp.bfloat16)
    f0be = fc0_be.reshape(1, 584)
    f0wr = fc0_Wr.astype(jnp.bfloat16)
    f0br = fc0_br.reshape(1, 4)
    f1w = jnp.transpose(fc1_We, (1, 0, 2)).reshape(146, 400).astype(jnp.bfloat16)
    f1be = fc1_be.reshape(1, 400)
    f1wr = fc1_Wr.astype(jnp.bfloat16)
    f1br = fc1_br.reshape(1, 4)

    full = lambda i: (0, 0)
    out = pl.pallas_call(
        _net_kernel,
        grid=(B // BC,),
        in_specs=[
            pl.BlockSpec((BC, 34, 8, 18), lambda i: (i, 0, 0, 0)),
            pl.BlockSpec((54, 2048), full),
            pl.BlockSpec((1, 512), full),
            pl.BlockSpec((1, 4), full),
            pl.BlockSpec((1152, 772), full),
            pl.BlockSpec((1, 768), full),
            pl.BlockSpec((1, 4), full),
            pl.BlockSpec((192, 584), full),
            pl.BlockSpec((1, 584), full),
            pl.BlockSpec((192, 4), full),
            pl.BlockSpec((1, 4), full),
            pl.BlockSpec((146, 400), full),
            pl.BlockSpec((1, 400), full),
            pl.BlockSpec((146, 4), full),
            pl.BlockSpec((1, 4), full),
        ],
        out_specs=pl.BlockSpec((BC, 100), lambda i: (i, 0)),
        out_shape=jax.ShapeDtypeStruct((B, 100), jnp.float32),
        compiler_params=pltpu.CompilerParams(
            dimension_semantics=("arbitrary",)),
    )(xg, w0, be0, br0, w1, be1, br1,
      f0w, f0be, f0wr, f0br, f1w, f1be, f1wr, f1br)
    return out
